# Initial kernel scaffold; baseline (speedup 1.0000x reference)
import jax
import jax.numpy as jnp
from jax.experimental import pallas as pl


def kernel(x, edge_index, batch, W1, b1, W2, b2, Wg, bg):
    raise NotImplementedError("write your pallas kernel here")


# speedup vs baseline: 10.0951x; 10.0951x over previous
import functools

import jax
import jax.numpy as jnp
from jax import lax
from jax.experimental import pallas as pl
from jax.experimental.pallas import tpu as pltpu
from jax.experimental.pallas import tpu_sc as plsc

_N = 10000
_E = 320000
_B = 64
_NC = 2
_NS = 16
_NW = _NC * _NS
_EW = _E // _NW
_CHUNK = 80
_NSTEP = _EW // _CHUNK
_NPAD = 10240
_RPT = _NPAD // _NS

_BR = 2000
_NB = _N // _BR


def _sc_mesh():
    return plsc.VectorSubcoreMesh(core_axis_name="c", subcore_axis_name="s")


def _deg_kernel():
    @functools.partial(
        pl.kernel,
        out_type=jax.ShapeDtypeStruct((_NC, _NPAD, 128), jnp.float32),
        mesh=_sc_mesh(),
        scratch_types=[
            pltpu.VMEM((_CHUNK,), jnp.int32),
            pltpu.VMEM((_CHUNK, 128), jnp.float32),
            pltpu.VMEM((_CHUNK, 128), jnp.float32),
            pltpu.VMEM_SHARED((_NPAD, 128), jnp.float32),
        ],
    )
    def deg_k(dst_hbm, degp_hbm, idx_v, ones_v, zero_v, acc_sh):
        c = lax.axis_index("c")
        s = lax.axis_index("s")
        w = c * _NS + s

        def fill(r, carry):
            for j in range(8):
                ones_v[r, pl.ds(j * 16, 16)] = jnp.full((16,), 1.0, jnp.float32)
                zero_v[r, pl.ds(j * 16, 16)] = jnp.zeros((16,), jnp.float32)
            return carry

        lax.fori_loop(0, _CHUNK, fill, 0)

        def zero_step(j, carry):
            pltpu.sync_copy(zero_v, acc_sh.at[pl.ds(s * _RPT + j * _CHUNK, _CHUNK)])
            return carry

        lax.fori_loop(0, _RPT // _CHUNK, zero_step, 0)
        plsc.subcore_barrier()

        def step(i, carry):
            base = w * _EW + i * _CHUNK
            pltpu.sync_copy(dst_hbm.at[pl.ds(base, _CHUNK)], idx_v)
            pltpu.sync_copy(ones_v, acc_sh.at[idx_v], add=True)
            return carry

        lax.fori_loop(0, _NSTEP, step, 0)
        plsc.subcore_barrier()
        pltpu.sync_copy(acc_sh.at[pl.ds(s * _RPT, _RPT)],
                        degp_hbm.at[c, pl.ds(s * _RPT, _RPT)])

    return deg_k


def _scatter_kernel(C):
    @functools.partial(
        pl.kernel,
        out_type=jax.ShapeDtypeStruct((_NC, _NPAD, C), jnp.float32),
        mesh=_sc_mesh(),
        scratch_types=[
            pltpu.VMEM((_CHUNK,), jnp.int32),
            pltpu.VMEM((_CHUNK,), jnp.int32),
            pltpu.VMEM((_CHUNK, C), jnp.float32),
            pltpu.VMEM((_CHUNK, C), jnp.float32),
            pltpu.VMEM_SHARED((_NPAD, C), jnp.float32),
        ],
    )
    def scat_k(src_hbm, dst_hbm, y_hbm, part_hbm, sidx, didx, rows, zbuf, acc_sh):
        c = lax.axis_index("c")
        s = lax.axis_index("s")
        w = c * _NS + s

        def fill(r, carry):
            for j in range(C // 16):
                zbuf[r, pl.ds(j * 16, 16)] = jnp.zeros((16,), jnp.float32)
            return carry

        lax.fori_loop(0, _CHUNK, fill, 0)

        def zero_step(j, carry):
            pltpu.sync_copy(zbuf, acc_sh.at[pl.ds(s * _RPT + j * _CHUNK, _CHUNK)])
            return carry

        lax.fori_loop(0, _RPT // _CHUNK, zero_step, 0)
        plsc.subcore_barrier()

        def step(i, carry):
            base = w * _EW + i * _CHUNK
            pltpu.sync_copy(src_hbm.at[pl.ds(base, _CHUNK)], sidx)
            pltpu.sync_copy(dst_hbm.at[pl.ds(base, _CHUNK)], didx)
            pltpu.sync_copy(y_hbm.at[sidx], rows)
            pltpu.sync_copy(rows, acc_sh.at[didx], add=True)
            return carry

        lax.fori_loop(0, _NSTEP, step, 0)
        plsc.subcore_barrier()
        pltpu.sync_copy(acc_sh.at[pl.ds(s * _RPT, _RPT)],
                        part_hbm.at[c, pl.ds(s * _RPT, _RPT)])

    return scat_k


def _dis_from_degp(degp_blk):
    deg = degp_blk[0, :, :1] + degp_blk[1, :, :1] + 1.0
    return lax.rsqrt(deg)


def _y1_body(x_ref, w1_ref, degp_ref, y1_ref):
    dis = _dis_from_degp(degp_ref[...])
    xw = jnp.dot(x_ref[...], w1_ref[...], preferred_element_type=jnp.float32)
    y1_ref[...] = xw * dis


def _y2_body(y1_ref, p_ref, degp_ref, b1_ref, w2_ref, y2_ref):
    dis = _dis_from_degp(degp_ref[...])
    ssum = p_ref[0] + p_ref[1] + y1_ref[...]
    h = jnp.maximum(dis * ssum + b1_ref[...], 0.0)
    y2_ref[...] = jnp.dot(h, w2_ref[...], preferred_element_type=jnp.float32) * dis


def _pool_body(y2_ref, pa_ref, pb_ref, degp_ref, b2_ref, batch_ref, wg_ref,
               bg_ref, emb_ref, log_ref, emb_acc, cnt_acc):
    i = pl.program_id(0)

    @pl.when(i == 0)
    def _():
        emb_acc[...] = jnp.zeros_like(emb_acc)
        cnt_acc[...] = jnp.zeros_like(cnt_acc)

    dis = _dis_from_degp(degp_ref[...])
    ha = dis * (pa_ref[0] + pa_ref[1] + y2_ref[:, :128]) + b2_ref[:, :128]
    hb = dis * (pb_ref[0] + pb_ref[1] + y2_ref[:, 128:]) + b2_ref[:, 128:]
    h2 = jnp.concatenate([ha, hb], axis=1)

    bb = batch_ref[0, 0, :]
    gids = lax.broadcasted_iota(jnp.int32, (_B, _BR), 0)
    m = jnp.where(gids == bb[None, :], 1.0, 0.0)
    emb_acc[...] += jnp.dot(m, h2, preferred_element_type=jnp.float32)
    cnt_acc[...] += jnp.broadcast_to(
        jnp.sum(m, axis=1, keepdims=True), cnt_acc.shape)

    @pl.when(i == _NB - 1)
    def _():
        cnt = jnp.maximum(cnt_acc[:, :1], 1.0)
        emb = emb_acc[...] / cnt
        emb_ref[...] = emb
        log_ref[...] = (jnp.dot(emb, wg_ref[...],
                                preferred_element_type=jnp.float32)
                        + bg_ref[...])


def kernel(x, edge_index, batch, W1, b1, W2, b2, Wg, bg):
    f32 = jnp.float32
    src = edge_index[0]
    dst = edge_index[1]

    degp = _deg_kernel()(dst)

    y1 = pl.pallas_call(
        _y1_body,
        grid=(_NB,),
        in_specs=[
            pl.BlockSpec((_BR, 128), lambda i: (i, 0)),
            pl.BlockSpec((128, 128), lambda i: (0, 0)),
            pl.BlockSpec((_NC, _BR, 128), lambda i: (0, i, 0)),
        ],
        out_specs=pl.BlockSpec((_BR, 128), lambda i: (i, 0)),
        out_shape=jax.ShapeDtypeStruct((_N, 128), f32),
    )(x, W1, degp)

    p1 = _scatter_kernel(128)(src, dst, y1)

    y2 = pl.pallas_call(
        _y2_body,
        grid=(_NB,),
        in_specs=[
            pl.BlockSpec((_BR, 128), lambda i: (i, 0)),
            pl.BlockSpec((_NC, _BR, 128), lambda i: (0, i, 0)),
            pl.BlockSpec((_NC, _BR, 128), lambda i: (0, i, 0)),
            pl.BlockSpec((1, 128), lambda i: (0, 0)),
            pl.BlockSpec((128, 256), lambda i: (0, 0)),
        ],
        out_specs=pl.BlockSpec((_BR, 256), lambda i: (i, 0)),
        out_shape=jax.ShapeDtypeStruct((_N, 256), f32),
    )(y1, p1, degp, b1.reshape(1, 128), W2)

    y2a = y2[:, :128]
    y2b = y2[:, 128:]
    p2a = _scatter_kernel(128)(src, dst, y2a)
    p2b = _scatter_kernel(128)(src, dst, y2b)

    batch3 = batch.reshape(_NB, 1, _BR)
    emb, logits = pl.pallas_call(
        _pool_body,
        grid=(_NB,),
        in_specs=[
            pl.BlockSpec((_BR, 256), lambda i: (i, 0)),
            pl.BlockSpec((_NC, _BR, 128), lambda i: (0, i, 0)),
            pl.BlockSpec((_NC, _BR, 128), lambda i: (0, i, 0)),
            pl.BlockSpec((_NC, _BR, 128), lambda i: (0, i, 0)),
            pl.BlockSpec((1, 256), lambda i: (0, 0)),
            pl.BlockSpec((1, 1, _BR), lambda i: (i, 0, 0)),
            pl.BlockSpec((256, 16), lambda i: (0, 0)),
            pl.BlockSpec((1, 16), lambda i: (0, 0)),
        ],
        out_specs=[
            pl.BlockSpec((_B, 256), lambda i: (0, 0)),
            pl.BlockSpec((_B, 16), lambda i: (0, 0)),
        ],
        out_shape=[
            jax.ShapeDtypeStruct((_B, 256), f32),
            jax.ShapeDtypeStruct((_B, 16), f32),
        ],
        scratch_shapes=[
            pltpu.VMEM((_B, 256), f32),
            pltpu.VMEM((_B, 128), f32),
        ],
    )(y2, p2a, p2b, degp, b2.reshape(1, 256), batch3, Wg, bg.reshape(1, 16))

    return (emb, logits)


# speedup vs baseline: 17.2307x; 1.7068x over previous
import functools

import jax
import jax.numpy as jnp
from jax import lax
from jax.experimental import pallas as pl
from jax.experimental.pallas import tpu as pltpu
from jax.experimental.pallas import tpu_sc as plsc

_N = 10000
_E = 320000
_B = 64
_NC = 2
_NS = 16
_NW = _NC * _NS
_EW = _E // _NW
_CHUNK = 80
_NSTEP = _EW // _CHUNK
_NPAD = 10240
_RPT = _NPAD // _NS

_BR = 2000
_NB = _N // _BR


def _sc_mesh():
    return plsc.VectorSubcoreMesh(core_axis_name="c", subcore_axis_name="s")


def _deg_kernel():
    @functools.partial(
        pl.kernel,
        out_type=jax.ShapeDtypeStruct((_NC, _NPAD, 128), jnp.float32),
        mesh=_sc_mesh(),
        scratch_types=[
            pltpu.VMEM((_CHUNK,), jnp.int32),
            pltpu.VMEM((_CHUNK, 128), jnp.float32),
            pltpu.VMEM((_CHUNK, 128), jnp.float32),
            pltpu.VMEM_SHARED((_NPAD, 128), jnp.float32),
        ],
    )
    def deg_k(dst_hbm, degp_hbm, idx_v, ones_v, zero_v, acc_sh):
        c = lax.axis_index("c")
        s = lax.axis_index("s")
        w = c * _NS + s

        def fill(r, carry):
            for j in range(8):
                ones_v[r, pl.ds(j * 16, 16)] = jnp.full((16,), 1.0, jnp.float32)
                zero_v[r, pl.ds(j * 16, 16)] = jnp.zeros((16,), jnp.float32)
            return carry

        lax.fori_loop(0, _CHUNK, fill, 0)

        def zero_step(j, carry):
            pltpu.sync_copy(zero_v, acc_sh.at[pl.ds(s * _RPT + j * _CHUNK, _CHUNK)])
            return carry

        lax.fori_loop(0, _RPT // _CHUNK, zero_step, 0)
        plsc.subcore_barrier()

        def step(i, carry):
            base = w * _EW + i * _CHUNK
            pltpu.sync_copy(dst_hbm.at[pl.ds(base, _CHUNK)], idx_v)
            pltpu.sync_copy(ones_v, acc_sh.at[idx_v], add=True)
            return carry

        lax.fori_loop(0, _NSTEP, step, 0)
        plsc.subcore_barrier()
        pltpu.sync_copy(acc_sh.at[pl.ds(s * _RPT, _RPT)],
                        degp_hbm.at[c, pl.ds(s * _RPT, _RPT)])

    return deg_k


_NSLOT = 4


def _scatter_kernel(nt):
    C = 128

    @functools.partial(
        pl.kernel,
        out_type=jax.ShapeDtypeStruct((nt, _NC, _NPAD, C), jnp.float32),
        mesh=_sc_mesh(),
        scratch_types=[
            pltpu.VMEM((_NSLOT, _CHUNK), jnp.int32),
            pltpu.VMEM((_NSLOT, _CHUNK), jnp.int32),
            pltpu.VMEM((_NSLOT, _CHUNK, C), jnp.float32),
            pltpu.VMEM_SHARED((_NPAD, C), jnp.float32),
            [pltpu.SemaphoreType.DMA] * _NSLOT,
            [pltpu.SemaphoreType.DMA] * _NSLOT,
            [pltpu.SemaphoreType.DMA] * _NSLOT,
        ],
    )
    def scat_k(src_hbm, dst_hbm, *rest):
        ys = rest[:nt]
        part_hbm = rest[nt]
        sidx, didx, rows, acc_sh, isems, gsems, ssems = rest[nt + 1:]
        c = lax.axis_index("c")
        s = lax.axis_index("s")
        w = c * _NS + s

        def fire_i(i, q):
            pltpu.async_copy(src_hbm.at[w, i], sidx.at[q], isems[q])
            pltpu.async_copy(dst_hbm.at[w, i], didx.at[q], isems[q])

        def wait_i(q):
            pltpu.make_async_copy(src_hbm.at[w, 0], sidx.at[q],
                                  isems[q]).wait()
            pltpu.make_async_copy(dst_hbm.at[w, 0], didx.at[q],
                                  isems[q]).wait()

        def fire_s(b):
            pltpu.async_copy(rows.at[b], acc_sh.at[didx.at[b]], ssems[b],
                             add=True)

        def wait_s(b):
            pltpu.make_async_copy(rows.at[b], acc_sh.at[pl.ds(0, _CHUNK)],
                                  ssems[b]).wait()

        for t in range(nt):
            y_hbm = ys[t]

            def fire_g(b):
                pltpu.async_copy(y_hbm.at[sidx.at[b]], rows.at[b], gsems[b])

            def wait_g(b):
                pltpu.make_async_copy(y_hbm.at[pl.ds(0, _CHUNK)], rows.at[b],
                                      gsems[b]).wait()

            def fillz(r, carry):
                for j in range(C // 16):
                    rows[0, r, pl.ds(j * 16, 16)] = jnp.zeros((16,),
                                                              jnp.float32)
                return carry

            lax.fori_loop(0, _CHUNK, fillz, 0)

            def zero_step(j, carry):
                pltpu.sync_copy(rows.at[0],
                                acc_sh.at[pl.ds(s * _RPT + j * _CHUNK,
                                                _CHUNK)])
                return carry

            lax.fori_loop(0, _RPT // _CHUNK, zero_step, 0)
            plsc.subcore_barrier()

            for q in range(_NSLOT):
                fire_i(q, q)
            wait_i(0)
            fire_g(0)
            for i in range(3):
                b = i % _NSLOT
                wait_g(b)
                fire_s(b)
                wait_i(b + 1)
                fire_g(b + 1)
            wait_s(0)
            fire_i(4, 0)
            wait_g(3)
            fire_s(3)
            wait_i(0)
            fire_g(0)

            def body(g, carry):
                for b in range(_NSLOT):
                    i = g * _NSLOT + b
                    bn = (b + 1) % _NSLOT
                    wait_s(bn)
                    fire_i(i + 1, bn)
                    wait_g(b)
                    fire_s(b)
                    wait_i(bn)
                    fire_g(bn)
                return carry

            lax.fori_loop(1, _NSTEP // _NSLOT, body, 0)

            lb = (_NSTEP - 1) % _NSLOT
            wait_g(lb)
            fire_s(lb)
            for b in range(_NSLOT):
                wait_s(b)

            plsc.subcore_barrier()
            pltpu.sync_copy(acc_sh.at[pl.ds(s * _RPT, _RPT)],
                            part_hbm.at[t, c, pl.ds(s * _RPT, _RPT)])
            plsc.subcore_barrier()

    return scat_k


def _dis_from_degp(degp_blk):
    deg = degp_blk[0, :, :1] + degp_blk[1, :, :1] + 1.0
    return lax.rsqrt(deg)


def _y1_body(x_ref, w1_ref, degp_ref, y1_ref):
    dis = _dis_from_degp(degp_ref[...])
    xw = jnp.dot(x_ref[...], w1_ref[...], preferred_element_type=jnp.float32)
    y1_ref[...] = xw * dis


def _y2_body(y1_ref, p_ref, degp_ref, b1_ref, w2_ref, y2_ref):
    dis = _dis_from_degp(degp_ref[...])
    ssum = p_ref[0, 0] + p_ref[0, 1] + y1_ref[...]
    h = jnp.maximum(dis * ssum + b1_ref[...], 0.0)
    y2_ref[...] = jnp.dot(h, w2_ref[...], preferred_element_type=jnp.float32) * dis


def _pool_body(y2_ref, p2_ref, degp_ref, b2_ref, batch_ref, wg_ref,
               bg_ref, emb_ref, log_ref, emb_acc, cnt_acc):
    i = pl.program_id(0)

    @pl.when(i == 0)
    def _():
        emb_acc[...] = jnp.zeros_like(emb_acc)
        cnt_acc[...] = jnp.zeros_like(cnt_acc)

    dis = _dis_from_degp(degp_ref[...])
    ha = dis * (p2_ref[0, 0] + p2_ref[0, 1] + y2_ref[:, :128]) + b2_ref[:, :128]
    hb = dis * (p2_ref[1, 0] + p2_ref[1, 1] + y2_ref[:, 128:]) + b2_ref[:, 128:]
    h2 = jnp.concatenate([ha, hb], axis=1)

    bb = batch_ref[0, 0, :]
    gids = lax.broadcasted_iota(jnp.int32, (_B, _BR), 0)
    m = jnp.where(gids == bb[None, :], 1.0, 0.0)
    emb_acc[...] += jnp.dot(m, h2, preferred_element_type=jnp.float32)
    cnt_acc[...] += jnp.broadcast_to(
        jnp.sum(m, axis=1, keepdims=True), cnt_acc.shape)

    @pl.when(i == _NB - 1)
    def _():
        cnt = jnp.maximum(cnt_acc[:, :1], 1.0)
        emb = emb_acc[...] / cnt
        emb_ref[...] = emb
        log_ref[...] = (jnp.dot(emb, wg_ref[...],
                                preferred_element_type=jnp.float32)
                        + bg_ref[...])


def kernel(x, edge_index, batch, W1, b1, W2, b2, Wg, bg):
    f32 = jnp.float32
    src = edge_index[0]
    dst = edge_index[1]
    src2 = src.reshape(_NW, _NSTEP, _CHUNK)
    dst2 = dst.reshape(_NW, _NSTEP, _CHUNK)

    degp = _deg_kernel()(dst)

    y1 = pl.pallas_call(
        _y1_body,
        grid=(_NB,),
        in_specs=[
            pl.BlockSpec((_BR, 128), lambda i: (i, 0)),
            pl.BlockSpec((128, 128), lambda i: (0, 0)),
            pl.BlockSpec((_NC, _BR, 128), lambda i: (0, i, 0)),
        ],
        out_specs=pl.BlockSpec((_BR, 128), lambda i: (i, 0)),
        out_shape=jax.ShapeDtypeStruct((_N, 128), f32),
    )(x, W1, degp)

    p1 = _scatter_kernel(1)(src2, dst2, y1)

    y2 = pl.pallas_call(
        _y2_body,
        grid=(_NB,),
        in_specs=[
            pl.BlockSpec((_BR, 128), lambda i: (i, 0)),
            pl.BlockSpec((1, _NC, _BR, 128), lambda i: (0, 0, i, 0)),
            pl.BlockSpec((_NC, _BR, 128), lambda i: (0, i, 0)),
            pl.BlockSpec((1, 128), lambda i: (0, 0)),
            pl.BlockSpec((128, 256), lambda i: (0, 0)),
        ],
        out_specs=pl.BlockSpec((_BR, 256), lambda i: (i, 0)),
        out_shape=jax.ShapeDtypeStruct((_N, 256), f32),
    )(y1, p1, degp, b1.reshape(1, 128), W2)

    y2a = y2[:, :128]
    y2b = y2[:, 128:]
    p2 = _scatter_kernel(2)(src2, dst2, y2a, y2b)

    batch3 = batch.reshape(_NB, 1, _BR)
    emb, logits = pl.pallas_call(
        _pool_body,
        grid=(_NB,),
        in_specs=[
            pl.BlockSpec((_BR, 256), lambda i: (i, 0)),
            pl.BlockSpec((2, _NC, _BR, 128), lambda i: (0, 0, i, 0)),
            pl.BlockSpec((_NC, _BR, 128), lambda i: (0, i, 0)),
            pl.BlockSpec((1, 256), lambda i: (0, 0)),
            pl.BlockSpec((1, 1, _BR), lambda i: (i, 0, 0)),
            pl.BlockSpec((256, 16), lambda i: (0, 0)),
            pl.BlockSpec((1, 16), lambda i: (0, 0)),
        ],
        out_specs=[
            pl.BlockSpec((_B, 256), lambda i: (0, 0)),
            pl.BlockSpec((_B, 16), lambda i: (0, 0)),
        ],
        out_shape=[
            jax.ShapeDtypeStruct((_B, 256), f32),
            jax.ShapeDtypeStruct((_B, 16), f32),
        ],
        scratch_shapes=[
            pltpu.VMEM((_B, 256), f32),
            pltpu.VMEM((_B, 128), f32),
        ],
    )(y2, p2, degp, b2.reshape(1, 256), batch3, Wg, bg.reshape(1, 16))

    return (emb, logits)


# speedup vs baseline: 22.3841x; 1.2991x over previous
import functools

import jax
import jax.numpy as jnp
from jax import lax
from jax.experimental import pallas as pl
from jax.experimental.pallas import tpu as pltpu
from jax.experimental.pallas import tpu_sc as plsc

_N = 10000
_E = 320000
_B = 64
_NC = 2
_NS = 16
_NW = _NC * _NS
_EW = _E // _NW
_CHUNK = 80
_NSTEP = _EW // _CHUNK
_NPAD = 10240
_RPT = _NPAD // _NS

_BR = 2000
_NB = _N // _BR


def _sc_mesh():
    return plsc.VectorSubcoreMesh(core_axis_name="c", subcore_axis_name="s")


def _deg_kernel():
    @functools.partial(
        pl.kernel,
        out_type=jax.ShapeDtypeStruct((_NC, _NPAD, 128), jnp.float32),
        mesh=_sc_mesh(),
        scratch_types=[
            pltpu.VMEM((_NSLOT, _CHUNK), jnp.int32),
            pltpu.VMEM((_CHUNK, 128), jnp.float32),
            pltpu.VMEM((_CHUNK, 128), jnp.float32),
            pltpu.VMEM_SHARED((_NPAD, 128), jnp.float32),
            [pltpu.SemaphoreType.DMA] * _NSLOT,
            [pltpu.SemaphoreType.DMA] * _NSLOT,
        ],
    )
    def deg_k(dst_hbm, degp_hbm, didx, ones_v, zero_v, acc_sh, isems, ssems):
        c = lax.axis_index("c")
        s = lax.axis_index("s")
        w = c * _NS + s

        def fill(r, carry):
            for j in range(8):
                ones_v[r, pl.ds(j * 16, 16)] = jnp.full((16,), 1.0, jnp.float32)
                zero_v[r, pl.ds(j * 16, 16)] = jnp.zeros((16,), jnp.float32)
            return carry

        lax.fori_loop(0, _CHUNK, fill, 0)

        def zero_step(j, carry):
            pltpu.sync_copy(zero_v, acc_sh.at[pl.ds(s * _RPT + j * _CHUNK, _CHUNK)])
            return carry

        lax.fori_loop(0, _RPT // _CHUNK, zero_step, 0)
        plsc.subcore_barrier()

        def fire_i(i, q):
            pltpu.async_copy(dst_hbm.at[w, i], didx.at[q], isems[q])

        def wait_i(q):
            pltpu.make_async_copy(dst_hbm.at[w, 0], didx.at[q],
                                  isems[q]).wait()

        def fire_s(q):
            pltpu.async_copy(ones_v, acc_sh.at[didx.at[q]], ssems[q],
                             add=True)

        def wait_s(q):
            pltpu.make_async_copy(ones_v, acc_sh.at[pl.ds(0, _CHUNK)],
                                  ssems[q]).wait()

        fire_i(0, 0)
        fire_i(1, 1)
        for i in range(4):
            wait_i(i)
            fire_s(i)
            if i >= 2:
                wait_s(i - 2)
            fire_i(i + 2, (i + 2) % _NSLOT)

        def body(g, carry):
            for b in range(_NSLOT):
                i = g * _NSLOT + b
                q2 = (b + 2) % _NSLOT
                wait_i(b)
                fire_s(b)
                wait_s(q2)
                nxt = jnp.minimum(i + 2, _NSTEP - 1)
                fire_i(nxt, q2)
            return carry

        lax.fori_loop(1, _NSTEP // _NSLOT, body, 0)

        last = _NSTEP - 1
        wait_i(last % _NSLOT)
        fire_s(last % _NSLOT)
        wait_i(_NSTEP % _NSLOT)
        wait_s((last - 2) % _NSLOT)
        wait_s((last - 1) % _NSLOT)
        wait_s(last % _NSLOT)
        plsc.subcore_barrier()
        pltpu.sync_copy(acc_sh.at[pl.ds(s * _RPT, _RPT)],
                        degp_hbm.at[c, pl.ds(s * _RPT, _RPT)])

    return deg_k


_NSLOT = 4


def _scatter_kernel(nt):
    C = 128

    @functools.partial(
        pl.kernel,
        out_type=jax.ShapeDtypeStruct((nt, _NC, _NPAD, C), jnp.float32),
        mesh=_sc_mesh(),
        scratch_types=[
            pltpu.VMEM((_NSLOT, _CHUNK), jnp.int32),
            pltpu.VMEM((_NSLOT, _CHUNK), jnp.int32),
            pltpu.VMEM((_NSLOT, _CHUNK, C), jnp.float32),
            pltpu.VMEM_SHARED((_NPAD, C), jnp.float32),
            [pltpu.SemaphoreType.DMA] * _NSLOT,
            [pltpu.SemaphoreType.DMA] * _NSLOT,
            [pltpu.SemaphoreType.DMA] * _NSLOT,
        ],
    )
    def scat_k(src_hbm, dst_hbm, *rest):
        ys = rest[:nt]
        part_hbm = rest[nt]
        sidx, didx, rows, acc_sh, isems, gsems, ssems = rest[nt + 1:]
        c = lax.axis_index("c")
        s = lax.axis_index("s")
        w = c * _NS + s

        def fire_i(i, q):
            pltpu.async_copy(src_hbm.at[w, i], sidx.at[q], isems[q])
            pltpu.async_copy(dst_hbm.at[w, i], didx.at[q], isems[q])

        def wait_i(q):
            pltpu.make_async_copy(src_hbm.at[w, 0], sidx.at[q],
                                  isems[q]).wait()
            pltpu.make_async_copy(dst_hbm.at[w, 0], didx.at[q],
                                  isems[q]).wait()

        def fire_s(b):
            pltpu.async_copy(rows.at[b], acc_sh.at[didx.at[b]], ssems[b],
                             add=True)

        def wait_s(b):
            pltpu.make_async_copy(rows.at[b], acc_sh.at[pl.ds(0, _CHUNK)],
                                  ssems[b]).wait()

        for t in range(nt):
            y_hbm = ys[t]

            def fire_g(b):
                pltpu.async_copy(y_hbm.at[sidx.at[b]], rows.at[b], gsems[b])

            def wait_g(b):
                pltpu.make_async_copy(y_hbm.at[pl.ds(0, _CHUNK)], rows.at[b],
                                      gsems[b]).wait()

            def fillz(r, carry):
                for j in range(C // 16):
                    rows[0, r, pl.ds(j * 16, 16)] = jnp.zeros((16,),
                                                              jnp.float32)
                return carry

            lax.fori_loop(0, _CHUNK, fillz, 0)

            def zero_step(j, carry):
                pltpu.sync_copy(rows.at[0],
                                acc_sh.at[pl.ds(s * _RPT + j * _CHUNK,
                                                _CHUNK)])
                return carry

            lax.fori_loop(0, _RPT // _CHUNK, zero_step, 0)
            plsc.subcore_barrier()

            for q in range(_NSLOT):
                fire_i(q, q)
            wait_i(0)
            fire_g(0)
            for i in range(3):
                b = i % _NSLOT
                wait_g(b)
                fire_s(b)
                wait_i(b + 1)
                fire_g(b + 1)
            wait_s(0)
            fire_i(4, 0)
            wait_g(3)
            fire_s(3)
            wait_i(0)
            fire_g(0)

            def body(g, carry):
                for b in range(_NSLOT):
                    i = g * _NSLOT + b
                    bn = (b + 1) % _NSLOT
                    wait_s(bn)
                    fire_i(i + 1, bn)
                    wait_g(b)
                    fire_s(b)
                    wait_i(bn)
                    fire_g(bn)
                return carry

            lax.fori_loop(1, _NSTEP // _NSLOT, body, 0)

            lb = (_NSTEP - 1) % _NSLOT
            wait_g(lb)
            fire_s(lb)
            for b in range(_NSLOT):
                wait_s(b)

            plsc.subcore_barrier()
            pltpu.sync_copy(acc_sh.at[pl.ds(s * _RPT, _RPT)],
                            part_hbm.at[t, c, pl.ds(s * _RPT, _RPT)])
            plsc.subcore_barrier()

    return scat_k


def _dis_from_degp(degp_blk):
    deg = degp_blk[0, :, :1] + degp_blk[1, :, :1] + 1.0
    return lax.rsqrt(deg)


def _y1_body(x_ref, w1_ref, degp_ref, y1_ref):
    dis = _dis_from_degp(degp_ref[...])
    xw = jnp.dot(x_ref[...], w1_ref[...], preferred_element_type=jnp.float32)
    y1_ref[...] = xw * dis


def _y2_body(y1_ref, p_ref, degp_ref, b1_ref, w2_ref, y2_ref):
    dis = _dis_from_degp(degp_ref[...])
    ssum = p_ref[0, 0] + p_ref[0, 1] + y1_ref[...]
    h = jnp.maximum(dis * ssum + b1_ref[...], 0.0)
    y2_ref[...] = jnp.dot(h, w2_ref[...], preferred_element_type=jnp.float32) * dis


def _pool_body(y2_ref, p2_ref, degp_ref, b2_ref, batch_ref, wg_ref,
               bg_ref, emb_ref, log_ref, emb_acc, cnt_acc):
    i = pl.program_id(0)

    @pl.when(i == 0)
    def _():
        emb_acc[...] = jnp.zeros_like(emb_acc)
        cnt_acc[...] = jnp.zeros_like(cnt_acc)

    dis = _dis_from_degp(degp_ref[...])
    ha = dis * (p2_ref[0, 0] + p2_ref[0, 1] + y2_ref[:, :128]) + b2_ref[:, :128]
    hb = dis * (p2_ref[1, 0] + p2_ref[1, 1] + y2_ref[:, 128:]) + b2_ref[:, 128:]
    h2 = jnp.concatenate([ha, hb], axis=1)

    bb = batch_ref[0, 0, :]
    gids = lax.broadcasted_iota(jnp.int32, (_B, _BR), 0)
    m = jnp.where(gids == bb[None, :], 1.0, 0.0)
    emb_acc[...] += jnp.dot(m, h2, preferred_element_type=jnp.float32)
    cnt_acc[...] += jnp.broadcast_to(
        jnp.sum(m, axis=1, keepdims=True), cnt_acc.shape)

    @pl.when(i == _NB - 1)
    def _():
        cnt = jnp.maximum(cnt_acc[:, :1], 1.0)
        emb = emb_acc[...] / cnt
        emb_ref[...] = emb
        log_ref[...] = (jnp.dot(emb, wg_ref[...],
                                preferred_element_type=jnp.float32)
                        + bg_ref[...])


def kernel(x, edge_index, batch, W1, b1, W2, b2, Wg, bg):
    f32 = jnp.float32
    src = edge_index[0]
    dst = edge_index[1]
    src2 = src.reshape(_NW, _NSTEP, _CHUNK)
    dst2 = dst.reshape(_NW, _NSTEP, _CHUNK)

    degp = _deg_kernel()(dst2)

    y1 = pl.pallas_call(
        _y1_body,
        grid=(_NB,),
        in_specs=[
            pl.BlockSpec((_BR, 128), lambda i: (i, 0)),
            pl.BlockSpec((128, 128), lambda i: (0, 0)),
            pl.BlockSpec((_NC, _BR, 128), lambda i: (0, i, 0)),
        ],
        out_specs=pl.BlockSpec((_BR, 128), lambda i: (i, 0)),
        out_shape=jax.ShapeDtypeStruct((_N, 128), f32),
    )(x, W1, degp)

    p1 = _scatter_kernel(1)(src2, dst2, y1)

    y2 = pl.pallas_call(
        _y2_body,
        grid=(_NB,),
        in_specs=[
            pl.BlockSpec((_BR, 128), lambda i: (i, 0)),
            pl.BlockSpec((1, _NC, _BR, 128), lambda i: (0, 0, i, 0)),
            pl.BlockSpec((_NC, _BR, 128), lambda i: (0, i, 0)),
            pl.BlockSpec((1, 128), lambda i: (0, 0)),
            pl.BlockSpec((128, 256), lambda i: (0, 0)),
        ],
        out_specs=pl.BlockSpec((_BR, 256), lambda i: (i, 0)),
        out_shape=jax.ShapeDtypeStruct((_N, 256), f32),
    )(y1, p1, degp, b1.reshape(1, 128), W2)

    y2a = y2[:, :128]
    y2b = y2[:, 128:]
    p2 = _scatter_kernel(2)(src2, dst2, y2a, y2b)

    batch3 = batch.reshape(_NB, 1, _BR)
    emb, logits = pl.pallas_call(
        _pool_body,
        grid=(_NB,),
        in_specs=[
            pl.BlockSpec((_BR, 256), lambda i: (i, 0)),
            pl.BlockSpec((2, _NC, _BR, 128), lambda i: (0, 0, i, 0)),
            pl.BlockSpec((_NC, _BR, 128), lambda i: (0, i, 0)),
            pl.BlockSpec((1, 256), lambda i: (0, 0)),
            pl.BlockSpec((1, 1, _BR), lambda i: (i, 0, 0)),
            pl.BlockSpec((256, 16), lambda i: (0, 0)),
            pl.BlockSpec((1, 16), lambda i: (0, 0)),
        ],
        out_specs=[
            pl.BlockSpec((_B, 256), lambda i: (0, 0)),
            pl.BlockSpec((_B, 16), lambda i: (0, 0)),
        ],
        out_shape=[
            jax.ShapeDtypeStruct((_B, 256), f32),
            jax.ShapeDtypeStruct((_B, 16), f32),
        ],
        scratch_shapes=[
            pltpu.VMEM((_B, 256), f32),
            pltpu.VMEM((_B, 128), f32),
        ],
    )(y2, p2, degp, b2.reshape(1, 256), batch3, Wg, bg.reshape(1, 16))

    return (emb, logits)


# speedup vs baseline: 25.8655x; 1.1555x over previous
import functools

import jax
import jax.numpy as jnp
from jax import lax
from jax.experimental import pallas as pl
from jax.experimental.pallas import tpu as pltpu
from jax.experimental.pallas import tpu_sc as plsc

_N = 10000
_E = 320000
_B = 64
_NC = 2
_NS = 16
_NW = _NC * _NS
_EW = _E // _NW
_CHUNK = 80
_NSTEP = _EW // _CHUNK
_NPAD = 10240
_RPT = _NPAD // _NS

_BR = 2000
_NB = _N // _BR


def _sc_mesh():
    return plsc.VectorSubcoreMesh(core_axis_name="c", subcore_axis_name="s")


def _deg_kernel():
    @functools.partial(
        pl.kernel,
        out_type=jax.ShapeDtypeStruct((_NC, _NPAD, 128), jnp.float32),
        mesh=_sc_mesh(),
        scratch_types=[
            pltpu.VMEM((_NSLOT, _CHUNK), jnp.int32),
            pltpu.VMEM((_CHUNK, 128), jnp.float32),
            pltpu.VMEM((_CHUNK, 128), jnp.float32),
            pltpu.VMEM_SHARED((_NPAD, 128), jnp.float32),
            [pltpu.SemaphoreType.DMA] * _NSLOT,
            [pltpu.SemaphoreType.DMA] * _NSLOT,
        ],
    )
    def deg_k(dst_hbm, degp_hbm, didx, ones_v, zero_v, acc_sh, isems, ssems):
        c = lax.axis_index("c")
        s = lax.axis_index("s")
        w = c * _NS + s

        def fill(r, carry):
            for j in range(8):
                ones_v[r, pl.ds(j * 16, 16)] = jnp.full((16,), 1.0, jnp.float32)
                zero_v[r, pl.ds(j * 16, 16)] = jnp.zeros((16,), jnp.float32)
            return carry

        lax.fori_loop(0, _CHUNK, fill, 0)

        def zero_step(j, carry):
            pltpu.sync_copy(zero_v, acc_sh.at[pl.ds(s * _RPT + j * _CHUNK, _CHUNK)])
            return carry

        lax.fori_loop(0, _RPT // _CHUNK, zero_step, 0)
        plsc.subcore_barrier()

        def fire_i(i, q):
            pltpu.async_copy(dst_hbm.at[w, i], didx.at[q], isems[q])

        def wait_i(q):
            pltpu.make_async_copy(dst_hbm.at[w, 0], didx.at[q],
                                  isems[q]).wait()

        def fire_s(q):
            pltpu.async_copy(ones_v, acc_sh.at[didx.at[q]], ssems[q],
                             add=True)

        def wait_s(q):
            pltpu.make_async_copy(ones_v, acc_sh.at[pl.ds(0, _CHUNK)],
                                  ssems[q]).wait()

        fire_i(0, 0)
        fire_i(1, 1)
        for i in range(4):
            wait_i(i)
            fire_s(i)
            if i >= 2:
                wait_s(i - 2)
            fire_i(i + 2, (i + 2) % _NSLOT)

        def body(g, carry):
            for b in range(_NSLOT):
                i = g * _NSLOT + b
                q2 = (b + 2) % _NSLOT
                wait_i(b)
                fire_s(b)
                wait_s(q2)
                nxt = jnp.minimum(i + 2, _NSTEP - 1)
                fire_i(nxt, q2)
            return carry

        lax.fori_loop(1, _NSTEP // _NSLOT, body, 0)

        last = _NSTEP - 1
        wait_i(last % _NSLOT)
        fire_s(last % _NSLOT)
        wait_i(_NSTEP % _NSLOT)
        wait_s((last - 2) % _NSLOT)
        wait_s((last - 1) % _NSLOT)
        wait_s(last % _NSLOT)
        plsc.subcore_barrier()
        pltpu.sync_copy(acc_sh.at[pl.ds(s * _RPT, _RPT)],
                        degp_hbm.at[c, pl.ds(s * _RPT, _RPT)])

    return deg_k


_NSLOT = 4


def _scatter_kernel(nt):
    C = 128
    KS = 62

    @functools.partial(
        pl.kernel,
        out_type=jax.ShapeDtypeStruct((nt, _NC, _NPAD, C), jnp.float32),
        mesh=_sc_mesh(),
        scratch_types=[
            pltpu.VMEM((4, _CHUNK), jnp.int32),
            pltpu.VMEM((4, _CHUNK), jnp.int32),
            pltpu.VMEM((4, _CHUNK), jnp.int32),
            pltpu.VMEM((4, _CHUNK), jnp.int32),
            pltpu.VMEM((4, _CHUNK, C), jnp.float32),
            pltpu.VMEM_SHARED((_NPAD, C), jnp.float32),
            [pltpu.SemaphoreType.DMA] * 4,
            [pltpu.SemaphoreType.DMA] * 4,
            [pltpu.SemaphoreType.DMA] * 4,
            [pltpu.SemaphoreType.DMA] * 4,
        ],
    )
    def scat_k(src_hbm, dst_hbm, *rest):
        ys = rest[:nt]
        part_hbm = rest[nt]
        (sidx_a, didx_a, sidx_b, didx_b, rows, acc_sh,
         isems_a, isems_b, gsems, ssems) = rest[nt + 1:]
        sidxs = (sidx_a, sidx_b)
        didxs = (didx_a, didx_b)
        isemss = (isems_a, isems_b)
        c = lax.axis_index("c")
        s = lax.axis_index("s")
        w = c * _NS + s

        streams = ((0, 0, 0), (KS, 2, 1))

        def fire_i(st, gi, q):
            base = w * _EW + gi * _CHUNK
            pltpu.async_copy(src_hbm.at[pl.ds(base, _CHUNK)],
                             sidxs[st].at[q], isemss[st][q])
            pltpu.async_copy(dst_hbm.at[pl.ds(base, _CHUNK)],
                             didxs[st].at[q], isemss[st][q])

        def wait_i(st, q):
            pltpu.make_async_copy(src_hbm.at[pl.ds(0, _CHUNK)],
                                  sidxs[st].at[q], isemss[st][q]).wait()
            pltpu.make_async_copy(dst_hbm.at[pl.ds(0, _CHUNK)],
                                  didxs[st].at[q], isemss[st][q]).wait()

        def fire_s(st, q, b):
            pltpu.async_copy(rows.at[b], acc_sh.at[didxs[st].at[q]],
                             ssems[b], add=True)

        def wait_s(b):
            pltpu.make_async_copy(rows.at[b], acc_sh.at[pl.ds(0, _CHUNK)],
                                  ssems[b]).wait()

        for t in range(nt):
            y_hbm = ys[t]

            def fire_g(st, q, b):
                pltpu.async_copy(y_hbm.at[sidxs[st].at[q]], rows.at[b],
                                 gsems[b])

            def wait_g(b):
                pltpu.make_async_copy(y_hbm.at[pl.ds(0, _CHUNK)], rows.at[b],
                                      gsems[b]).wait()

            def chunk_ops(ro, st, db, inext_g=None, gather_next=True,
                          skip_ws=False):
                b = ro + (db % 2)
                bn = ro + ((db + 1) % 2)
                q = db % 4
                qn = (db + 1) % 4
                qp = (db + 2) % 4
                if not skip_ws:
                    wait_s(bn)
                if inext_g is not None:
                    fire_i(st, inext_g, qp)
                if gather_next:
                    wait_i(st, qn)
                    fire_g(st, qn, bn)
                wait_g(b)
                fire_s(st, q, b)

            def fillz(r, carry):
                for j in range(C // 16):
                    rows[0, r, pl.ds(j * 16, 16)] = jnp.zeros((16,),
                                                              jnp.float32)
                return carry

            lax.fori_loop(0, _CHUNK, fillz, 0)

            def zero_step(j, carry):
                pltpu.sync_copy(rows.at[0],
                                acc_sh.at[pl.ds(s * _RPT + j * _CHUNK,
                                                _CHUNK)])
                return carry

            lax.fori_loop(0, _RPT // _CHUNK, zero_step, 0)
            plsc.subcore_barrier()

            for cb, ro, st in streams:
                fire_i(st, cb + 0, 0)
                fire_i(st, cb + 1, 1)
            for cb, ro, st in streams:
                wait_i(st, 0)
                fire_g(st, 0, ro + 0)
            for k in range(4):
                for cb, ro, st in streams:
                    chunk_ops(ro, st, k, inext_g=cb + k + 2,
                              skip_ws=(k == 0))

            def body(g, carry):
                for db in range(4):
                    k = g * 4 + db
                    for cb, ro, st in streams:
                        chunk_ops(ro, st, db, inext_g=cb + k + 2)
                return carry

            lax.fori_loop(1, (KS - 2) // 4, body, 0)

            chunk_ops(0, 0, 0, inext_g=_NSTEP - 1)
            chunk_ops(2, 1, 0, inext_g=None)
            chunk_ops(0, 0, 1, inext_g=None)
            chunk_ops(2, 1, 1, inext_g=None, gather_next=False)
            chunk_ops(0, 0, 2, inext_g=None, gather_next=False)
            wait_s(0)
            wait_s(3)

            plsc.subcore_barrier()
            pltpu.sync_copy(acc_sh.at[pl.ds(s * _RPT, _RPT)],
                            part_hbm.at[t, c, pl.ds(s * _RPT, _RPT)])
            plsc.subcore_barrier()

    return scat_k


def _dis_from_degp(degp_blk):
    deg = degp_blk[0, :, :1] + degp_blk[1, :, :1] + 1.0
    return lax.rsqrt(deg)


def _y1_body(x_ref, w1_ref, degp_ref, y1_ref):
    dis = _dis_from_degp(degp_ref[...])
    xw = jnp.dot(x_ref[...], w1_ref[...], preferred_element_type=jnp.float32)
    y1_ref[...] = xw * dis


def _y2_body(y1_ref, p_ref, degp_ref, b1_ref, w2_ref, y2_ref):
    dis = _dis_from_degp(degp_ref[...])
    ssum = p_ref[0, 0] + p_ref[0, 1] + y1_ref[...]
    h = jnp.maximum(dis * ssum + b1_ref[...], 0.0)
    y2_ref[...] = jnp.dot(h, w2_ref[...], preferred_element_type=jnp.float32) * dis


def _pool_body(y2_ref, p2_ref, degp_ref, b2_ref, batch_ref, wg_ref,
               bg_ref, emb_ref, log_ref, emb_acc, cnt_acc):
    i = pl.program_id(0)

    @pl.when(i == 0)
    def _():
        emb_acc[...] = jnp.zeros_like(emb_acc)
        cnt_acc[...] = jnp.zeros_like(cnt_acc)

    dis = _dis_from_degp(degp_ref[...])
    ha = dis * (p2_ref[0, 0] + p2_ref[0, 1] + y2_ref[:, :128]) + b2_ref[:, :128]
    hb = dis * (p2_ref[1, 0] + p2_ref[1, 1] + y2_ref[:, 128:]) + b2_ref[:, 128:]
    h2 = jnp.concatenate([ha, hb], axis=1)

    bb = batch_ref[0, 0, :]
    gids = lax.broadcasted_iota(jnp.int32, (_B, _BR), 0)
    m = jnp.where(gids == bb[None, :], 1.0, 0.0)
    emb_acc[...] += jnp.dot(m, h2, preferred_element_type=jnp.float32)
    cnt_acc[...] += jnp.broadcast_to(
        jnp.sum(m, axis=1, keepdims=True), cnt_acc.shape)

    @pl.when(i == _NB - 1)
    def _():
        cnt = jnp.maximum(cnt_acc[:, :1], 1.0)
        emb = emb_acc[...] / cnt
        emb_ref[...] = emb
        log_ref[...] = (jnp.dot(emb, wg_ref[...],
                                preferred_element_type=jnp.float32)
                        + bg_ref[...])


def kernel(x, edge_index, batch, W1, b1, W2, b2, Wg, bg):
    f32 = jnp.float32
    src = edge_index[0]
    dst = edge_index[1]
    src2 = src.reshape(_NW, _NSTEP, _CHUNK)
    dst2 = dst.reshape(_NW, _NSTEP, _CHUNK)

    degp = _deg_kernel()(dst2)

    y1 = pl.pallas_call(
        _y1_body,
        grid=(_NB,),
        in_specs=[
            pl.BlockSpec((_BR, 128), lambda i: (i, 0)),
            pl.BlockSpec((128, 128), lambda i: (0, 0)),
            pl.BlockSpec((_NC, _BR, 128), lambda i: (0, i, 0)),
        ],
        out_specs=pl.BlockSpec((_BR, 128), lambda i: (i, 0)),
        out_shape=jax.ShapeDtypeStruct((_N, 128), f32),
    )(x, W1, degp)

    p1 = _scatter_kernel(1)(src, dst, y1)

    y2 = pl.pallas_call(
        _y2_body,
        grid=(_NB,),
        in_specs=[
            pl.BlockSpec((_BR, 128), lambda i: (i, 0)),
            pl.BlockSpec((1, _NC, _BR, 128), lambda i: (0, 0, i, 0)),
            pl.BlockSpec((_NC, _BR, 128), lambda i: (0, i, 0)),
            pl.BlockSpec((1, 128), lambda i: (0, 0)),
            pl.BlockSpec((128, 256), lambda i: (0, 0)),
        ],
        out_specs=pl.BlockSpec((_BR, 256), lambda i: (i, 0)),
        out_shape=jax.ShapeDtypeStruct((_N, 256), f32),
    )(y1, p1, degp, b1.reshape(1, 128), W2)

    y2a = y2[:, :128]
    y2b = y2[:, 128:]
    p2 = _scatter_kernel(2)(src, dst, y2a, y2b)

    batch3 = batch.reshape(_NB, 1, _BR)
    emb, logits = pl.pallas_call(
        _pool_body,
        grid=(_NB,),
        in_specs=[
            pl.BlockSpec((_BR, 256), lambda i: (i, 0)),
            pl.BlockSpec((2, _NC, _BR, 128), lambda i: (0, 0, i, 0)),
            pl.BlockSpec((_NC, _BR, 128), lambda i: (0, i, 0)),
            pl.BlockSpec((1, 256), lambda i: (0, 0)),
            pl.BlockSpec((1, 1, _BR), lambda i: (i, 0, 0)),
            pl.BlockSpec((256, 16), lambda i: (0, 0)),
            pl.BlockSpec((1, 16), lambda i: (0, 0)),
        ],
        out_specs=[
            pl.BlockSpec((_B, 256), lambda i: (0, 0)),
            pl.BlockSpec((_B, 16), lambda i: (0, 0)),
        ],
        out_shape=[
            jax.ShapeDtypeStruct((_B, 256), f32),
            jax.ShapeDtypeStruct((_B, 16), f32),
        ],
        scratch_shapes=[
            pltpu.VMEM((_B, 256), f32),
            pltpu.VMEM((_B, 128), f32),
        ],
    )(y2, p2, degp, b2.reshape(1, 256), batch3, Wg, bg.reshape(1, 16))

    return (emb, logits)


# speedup vs baseline: 26.3347x; 1.0181x over previous
import functools

import jax
import jax.numpy as jnp
from jax import lax
from jax.experimental import pallas as pl
from jax.experimental.pallas import tpu as pltpu
from jax.experimental.pallas import tpu_sc as plsc

_N = 10000
_E = 320000
_B = 64
_NC = 2
_NS = 16
_NW = _NC * _NS
_EW = _E // _NW
_CHUNK = 80
_NSTEP = _EW // _CHUNK
_NPAD = 10240
_RPT = _NPAD // _NS

_BR = 2000
_NB = _N // _BR


def _sc_mesh():
    return plsc.VectorSubcoreMesh(core_axis_name="c", subcore_axis_name="s")


def _deg_kernel():
    @functools.partial(
        pl.kernel,
        out_type=jax.ShapeDtypeStruct((_NC, _NPAD, 128), jnp.float32),
        mesh=_sc_mesh(),
        scratch_types=[
            pltpu.VMEM((_NSLOT, _CHUNK), jnp.int32),
            pltpu.VMEM((_CHUNK, 128), jnp.float32),
            pltpu.VMEM((_CHUNK, 128), jnp.float32),
            pltpu.VMEM_SHARED((_NPAD, 128), jnp.float32),
            [pltpu.SemaphoreType.DMA] * _NSLOT,
            [pltpu.SemaphoreType.DMA] * _NSLOT,
        ],
    )
    def deg_k(dst_hbm, degp_hbm, didx, ones_v, zero_v, acc_sh, isems, ssems):
        c = lax.axis_index("c")
        s = lax.axis_index("s")
        w = c * _NS + s

        def fill(r, carry):
            for j in range(8):
                ones_v[r, pl.ds(j * 16, 16)] = jnp.full((16,), 1.0, jnp.float32)
                zero_v[r, pl.ds(j * 16, 16)] = jnp.zeros((16,), jnp.float32)
            return carry

        lax.fori_loop(0, _CHUNK, fill, 0)

        def zero_step(j, carry):
            pltpu.sync_copy(zero_v, acc_sh.at[pl.ds(s * _RPT + j * _CHUNK, _CHUNK)])
            return carry

        lax.fori_loop(0, _RPT // _CHUNK, zero_step, 0)
        plsc.subcore_barrier()

        def fire_i(i, q):
            pltpu.async_copy(dst_hbm.at[w, i], didx.at[q], isems[q])

        def wait_i(q):
            pltpu.make_async_copy(dst_hbm.at[w, 0], didx.at[q],
                                  isems[q]).wait()

        def fire_s(q):
            pltpu.async_copy(ones_v, acc_sh.at[didx.at[q]], ssems[q],
                             add=True)

        def wait_s(q):
            pltpu.make_async_copy(ones_v, acc_sh.at[pl.ds(0, _CHUNK)],
                                  ssems[q]).wait()

        fire_i(0, 0)
        fire_i(1, 1)
        for i in range(4):
            wait_i(i)
            fire_s(i)
            if i >= 2:
                wait_s(i - 2)
            fire_i(i + 2, (i + 2) % _NSLOT)

        def body(g, carry):
            for b in range(_NSLOT):
                i = g * _NSLOT + b
                q2 = (b + 2) % _NSLOT
                wait_i(b)
                fire_s(b)
                wait_s(q2)
                nxt = jnp.minimum(i + 2, _NSTEP - 1)
                fire_i(nxt, q2)
            return carry

        lax.fori_loop(1, _NSTEP // _NSLOT, body, 0)

        last = _NSTEP - 1
        wait_i(last % _NSLOT)
        fire_s(last % _NSLOT)
        wait_i(_NSTEP % _NSLOT)
        wait_s((last - 2) % _NSLOT)
        wait_s((last - 1) % _NSLOT)
        wait_s(last % _NSLOT)
        plsc.subcore_barrier()
        pltpu.sync_copy(acc_sh.at[pl.ds(s * _RPT, _RPT)],
                        degp_hbm.at[c, pl.ds(s * _RPT, _RPT)])

    return deg_k


_NSLOT = 4


def _scatter_kernel(nt):
    C = 128
    KS = 62

    @functools.partial(
        pl.kernel,
        out_type=jax.ShapeDtypeStruct((nt, _NC, _NPAD, C), jnp.float32),
        mesh=_sc_mesh(),
        scratch_types=[
            pltpu.VMEM((4, _CHUNK), jnp.int32),
            pltpu.VMEM((4, _CHUNK), jnp.int32),
            pltpu.VMEM((4, _CHUNK), jnp.int32),
            pltpu.VMEM((4, _CHUNK), jnp.int32),
            pltpu.VMEM((4, _CHUNK, C), jnp.float32),
            pltpu.VMEM_SHARED((_NPAD, C), jnp.float32),
            [pltpu.SemaphoreType.DMA] * 4,
            [pltpu.SemaphoreType.DMA] * 4,
            [pltpu.SemaphoreType.DMA] * 4,
            [pltpu.SemaphoreType.DMA] * 4,
        ],
    )
    def scat_k(src_hbm, dst_hbm, *rest):
        ys = rest[:nt]
        part_hbm = rest[nt]
        (sidx_a, didx_a, sidx_b, didx_b, rows, acc_sh,
         isems_a, isems_b, gsems, ssems) = rest[nt + 1:]
        sidxs = (sidx_a, sidx_b)
        didxs = (didx_a, didx_b)
        isemss = (isems_a, isems_b)
        c = lax.axis_index("c")
        s = lax.axis_index("s")
        w = c * _NS + s

        streams = ((0, 0, 0), (KS, 2, 1))

        def fire_i(st, gi, q):
            base = w * _EW + gi * _CHUNK
            pltpu.async_copy(src_hbm.at[pl.ds(base, _CHUNK)],
                             sidxs[st].at[q], isemss[st][q])
            pltpu.async_copy(dst_hbm.at[pl.ds(base, _CHUNK)],
                             didxs[st].at[q], isemss[st][q])

        def wait_i(st, q):
            pltpu.make_async_copy(src_hbm.at[pl.ds(0, _CHUNK)],
                                  sidxs[st].at[q], isemss[st][q]).wait()
            pltpu.make_async_copy(dst_hbm.at[pl.ds(0, _CHUNK)],
                                  didxs[st].at[q], isemss[st][q]).wait()

        def fire_s(st, q, b):
            pltpu.async_copy(rows.at[b], acc_sh.at[didxs[st].at[q]],
                             ssems[b], add=True)

        def wait_s(b):
            pltpu.make_async_copy(rows.at[b], acc_sh.at[pl.ds(0, _CHUNK)],
                                  ssems[b]).wait()

        for t in range(nt):
            y_hbm = ys[t]

            def fire_g(st, q, b):
                pltpu.async_copy(y_hbm.at[sidxs[st].at[q]], rows.at[b],
                                 gsems[b])

            def wait_g(b):
                pltpu.make_async_copy(y_hbm.at[pl.ds(0, _CHUNK)], rows.at[b],
                                      gsems[b]).wait()

            def chunk_ops(ro, st, db, inext_g=None, gather_next=True,
                          skip_ws=False):
                b = ro + (db % 2)
                bn = ro + ((db + 1) % 2)
                q = db % 4
                qn = (db + 1) % 4
                qp = (db + 2) % 4
                if not skip_ws:
                    wait_s(bn)
                if inext_g is not None:
                    fire_i(st, inext_g, qp)
                if gather_next:
                    wait_i(st, qn)
                    fire_g(st, qn, bn)
                wait_g(b)
                fire_s(st, q, b)

            def fillz(r, carry):
                for j in range(C // 16):
                    rows[0, r, pl.ds(j * 16, 16)] = jnp.zeros((16,),
                                                              jnp.float32)
                return carry

            lax.fori_loop(0, _CHUNK, fillz, 0)

            def zero_step(j, carry):
                pltpu.sync_copy(rows.at[0],
                                acc_sh.at[pl.ds(s * _RPT + j * _CHUNK,
                                                _CHUNK)])
                return carry

            lax.fori_loop(0, _RPT // _CHUNK, zero_step, 0)
            plsc.subcore_barrier()

            for cb, ro, st in streams:
                fire_i(st, cb + 0, 0)
                fire_i(st, cb + 1, 1)
            for cb, ro, st in streams:
                wait_i(st, 0)
                fire_g(st, 0, ro + 0)
            for k in range(4):
                for cb, ro, st in streams:
                    chunk_ops(ro, st, k, inext_g=cb + k + 2,
                              skip_ws=(k == 0))

            def body(g, carry):
                for db in range(4):
                    k = g * 4 + db
                    for cb, ro, st in streams:
                        chunk_ops(ro, st, db, inext_g=cb + k + 2)
                return carry

            lax.fori_loop(1, (KS - 2) // 4, body, 0)

            chunk_ops(0, 0, 0, inext_g=_NSTEP - 1)
            chunk_ops(2, 1, 0, inext_g=None)
            chunk_ops(0, 0, 1, inext_g=None)
            chunk_ops(2, 1, 1, inext_g=None, gather_next=False)
            chunk_ops(0, 0, 2, inext_g=None, gather_next=False)
            wait_s(0)
            wait_s(3)

            plsc.subcore_barrier()
            pltpu.sync_copy(acc_sh.at[pl.ds(s * _RPT, _RPT)],
                            part_hbm.at[t, c, pl.ds(s * _RPT, _RPT)])
            plsc.subcore_barrier()

    return scat_k


def _dis_from_degp(degp_blk):
    deg = degp_blk[0, :, :1] + degp_blk[1, :, :1] + 1.0
    return lax.rsqrt(deg)


def _y1_body(x_ref, w1_ref, degp_ref, y1_ref):
    dis = _dis_from_degp(degp_ref[...])
    xw = jnp.dot(x_ref[...], w1_ref[...], preferred_element_type=jnp.float32)
    y1_ref[...] = xw * dis


def _y2_body(y1_ref, p_ref, degp_ref, b1_ref, w2_ref, y2a_ref, y2b_ref):
    dis = _dis_from_degp(degp_ref[...])
    ssum = p_ref[0, 0] + p_ref[0, 1] + y1_ref[...]
    h = jnp.maximum(dis * ssum + b1_ref[...], 0.0)
    y2 = jnp.dot(h, w2_ref[...], preferred_element_type=jnp.float32) * dis
    y2a_ref[...] = y2[:, :128]
    y2b_ref[...] = y2[:, 128:]


def _pool_body(y2a_ref, y2b_ref, p2_ref, degp_ref, b2_ref, batch_ref, wg_ref,
               bg_ref, emb_ref, log_ref, emb_acc, cnt_acc):
    i = pl.program_id(0)

    @pl.when(i == 0)
    def _():
        emb_acc[...] = jnp.zeros_like(emb_acc)
        cnt_acc[...] = jnp.zeros_like(cnt_acc)

    dis = _dis_from_degp(degp_ref[...])
    ha = dis * (p2_ref[0, 0] + p2_ref[0, 1] + y2a_ref[...]) + b2_ref[:, :128]
    hb = dis * (p2_ref[1, 0] + p2_ref[1, 1] + y2b_ref[...]) + b2_ref[:, 128:]
    h2 = jnp.concatenate([ha, hb], axis=1)

    bb = batch_ref[0, 0, :]
    gids = lax.broadcasted_iota(jnp.int32, (_B, _BR), 0)
    m = jnp.where(gids == bb[None, :], 1.0, 0.0)
    emb_acc[...] += jnp.dot(m, h2, preferred_element_type=jnp.float32)
    cnt_acc[...] += jnp.broadcast_to(
        jnp.sum(m, axis=1, keepdims=True), cnt_acc.shape)

    @pl.when(i == _NB - 1)
    def _():
        cnt = jnp.maximum(cnt_acc[:, :1], 1.0)
        emb = emb_acc[...] / cnt
        emb_ref[...] = emb
        log_ref[...] = (jnp.dot(emb, wg_ref[...],
                                preferred_element_type=jnp.float32)
                        + bg_ref[...])


def kernel(x, edge_index, batch, W1, b1, W2, b2, Wg, bg):
    f32 = jnp.float32
    src = edge_index[0]
    dst = edge_index[1]
    src2 = src.reshape(_NW, _NSTEP, _CHUNK)
    dst2 = dst.reshape(_NW, _NSTEP, _CHUNK)

    degp = _deg_kernel()(dst2)

    y1 = pl.pallas_call(
        _y1_body,
        grid=(_NB,),
        in_specs=[
            pl.BlockSpec((_BR, 128), lambda i: (i, 0)),
            pl.BlockSpec((128, 128), lambda i: (0, 0)),
            pl.BlockSpec((_NC, _BR, 128), lambda i: (0, i, 0)),
        ],
        out_specs=pl.BlockSpec((_BR, 128), lambda i: (i, 0)),
        out_shape=jax.ShapeDtypeStruct((_N, 128), f32),
    )(x, W1, degp)

    p1 = _scatter_kernel(1)(src, dst, y1)

    y2a, y2b = pl.pallas_call(
        _y2_body,
        grid=(_NB,),
        in_specs=[
            pl.BlockSpec((_BR, 128), lambda i: (i, 0)),
            pl.BlockSpec((1, _NC, _BR, 128), lambda i: (0, 0, i, 0)),
            pl.BlockSpec((_NC, _BR, 128), lambda i: (0, i, 0)),
            pl.BlockSpec((1, 128), lambda i: (0, 0)),
            pl.BlockSpec((128, 256), lambda i: (0, 0)),
        ],
        out_specs=[
            pl.BlockSpec((_BR, 128), lambda i: (i, 0)),
            pl.BlockSpec((_BR, 128), lambda i: (i, 0)),
        ],
        out_shape=[
            jax.ShapeDtypeStruct((_N, 128), f32),
            jax.ShapeDtypeStruct((_N, 128), f32),
        ],
    )(y1, p1, degp, b1.reshape(1, 128), W2)

    p2 = _scatter_kernel(2)(src, dst, y2a, y2b)

    batch3 = batch.reshape(_NB, 1, _BR)
    emb, logits = pl.pallas_call(
        _pool_body,
        grid=(_NB,),
        in_specs=[
            pl.BlockSpec((_BR, 128), lambda i: (i, 0)),
            pl.BlockSpec((_BR, 128), lambda i: (i, 0)),
            pl.BlockSpec((2, _NC, _BR, 128), lambda i: (0, 0, i, 0)),
            pl.BlockSpec((_NC, _BR, 128), lambda i: (0, i, 0)),
            pl.BlockSpec((1, 256), lambda i: (0, 0)),
            pl.BlockSpec((1, 1, _BR), lambda i: (i, 0, 0)),
            pl.BlockSpec((256, 16), lambda i: (0, 0)),
            pl.BlockSpec((1, 16), lambda i: (0, 0)),
        ],
        out_specs=[
            pl.BlockSpec((_B, 256), lambda i: (0, 0)),
            pl.BlockSpec((_B, 16), lambda i: (0, 0)),
        ],
        out_shape=[
            jax.ShapeDtypeStruct((_B, 256), f32),
            jax.ShapeDtypeStruct((_B, 16), f32),
        ],
        scratch_shapes=[
            pltpu.VMEM((_B, 256), f32),
            pltpu.VMEM((_B, 128), f32),
        ],
    )(y2a, y2b, p2, degp, b2.reshape(1, 256), batch3, Wg, bg.reshape(1, 16))

    return (emb, logits)
